# baseline (speedup 1.0000x reference)
import functools

import jax
import jax.numpy as jnp
from jax import lax
from jax.experimental import pallas as pl
from jax.experimental.pallas import tpu as pltpu
from jax.experimental.pallas import tpu_sc as plsc

NC = 2
NS = 16
NW = NC * NS
CH = 128
R = 4
AH = 3
NB = 2

B_TOT = 16384
T_LEN = 50
DIM = 64
NCH = B_TOT * T_LEN // (NW * CH)


@jax.jit
def _sc_gather(table_pad, idx):
    mesh = plsc.VectorSubcoreMesh(core_axis_name="c", subcore_axis_name="s")

    @functools.partial(
        pl.kernel,
        mesh=mesh,
        out_type=jax.ShapeDtypeStruct((T_LEN, DIM, B_TOT), jnp.float32),
        compiler_params=pltpu.CompilerParams(
            use_tc_tiling_on_sc=True, needs_layout_passes=False),
        scratch_types=(
            [pltpu.VMEM((NCH, CH), jnp.int32)]
            + [pltpu.VMEM((CH, 128), jnp.float32) for _ in range(R)]
            + [pltpu.VMEM((DIM, CH), jnp.float32) for _ in range(NB)]
            + [pltpu.SemaphoreType.DMA for _ in range(R + NB)]
        ),
    )
    def k(table_hbm, idx_hbm, out_hbm, idx_v, *rest):
        gbufs = rest[:R]
        obufs = rest[R:R + NB]
        gsems = rest[R + NB:2 * R + NB]
        osems = rest[2 * R + NB:2 * R + 2 * NB]
        c = lax.axis_index("c")
        s = lax.axis_index("s")
        wid = s * NC + c
        m0 = wid * NCH
        pltpu.sync_copy(idx_hbm.at[wid], idx_v)
        for b in range(AH):
            pltpu.make_async_copy(
                table_hbm.at[idx_v.at[b]], gbufs[b], gsems[b]).start()

        iota16 = lax.iota(jnp.int32, 16)
        rows_g = [g * 16 + iota16 for g in range(8)]

        def transpose_block(gbuf, obuf):
            @plsc.parallel_loop(0, DIM, unroll=4)
            def _(f):
                cols = jnp.full((16,), 0, jnp.int32) + f
                for g in range(8):
                    v = plsc.load_gather(gbuf, [rows_g[g], cols])
                    obuf[f, pl.ds(g * 16, 16)] = v

        def consume(j, b, ob):
            m = m0 + j
            t = m // 128
            b0 = (m % 128) * 128
            pltpu.make_async_copy(
                table_hbm.at[idx_v.at[j]], gbufs[b], gsems[b]).wait()

            @pl.when(j + AH < NCH)
            def _():
                pltpu.make_async_copy(
                    table_hbm.at[idx_v.at[j + AH]], gbufs[(b + AH) % R],
                    gsems[(b + AH) % R]).start()

            @pl.when(j >= NB)
            def _():
                mo = m0 + j - NB
                pltpu.make_async_copy(
                    obufs[ob],
                    out_hbm.at[mo // 128, :,
                               pl.ds((mo % 128) * 128, CH)],
                    osems[ob]).wait()
            transpose_block(gbufs[b], obufs[ob])
            pltpu.make_async_copy(
                obufs[ob], out_hbm.at[t, :, pl.ds(b0, CH)], osems[ob]).start()

        nq = NCH // R

        def body2(q, _):
            j0 = q * R
            for b in range(R):
                consume(j0 + b, b, b % NB)
            return 0

        lax.fori_loop(0, nq, body2, 0)
        for ob in range(NB):
            mo = m0 + NCH - NB + ob
            pltpu.make_async_copy(
                obufs[ob % NB],
                out_hbm.at[mo // 128, :, pl.ds((mo % 128) * 128, CH)],
                osems[ob % NB]).wait()

    return k(table_pad, idx)


def kernel(x, table):
    table_pad = jnp.pad(table, ((0, 0), (0, 128 - DIM)))
    idx = x.T.reshape(NW, NCH, CH).astype(jnp.int32)
    out3 = _sc_gather(table_pad, idx)
    return jnp.transpose(out3, (2, 0, 1))


# speedup vs baseline: 1.3908x; 1.3908x over previous
import functools

import jax
import jax.numpy as jnp
from jax import lax
from jax.experimental import pallas as pl
from jax.experimental.pallas import tpu as pltpu
from jax.experimental.pallas import tpu_sc as plsc

NC = 2
NS = 16
NW = NC * NS
CH = 128
R = 4
AH = 3
NB = 2

B_TOT = 16384
T_LEN = 50
DIM = 64
NCH = B_TOT * T_LEN // (NW * CH)


@jax.jit
def _sc_gather(table_pad, idx):
    mesh = plsc.VectorSubcoreMesh(core_axis_name="c", subcore_axis_name="s")

    @functools.partial(
        pl.kernel,
        mesh=mesh,
        out_type=jax.ShapeDtypeStruct((T_LEN, DIM, B_TOT), jnp.float32),
        compiler_params=pltpu.CompilerParams(
            use_tc_tiling_on_sc=True, needs_layout_passes=False),
        scratch_types=(
            [pltpu.VMEM((NCH, CH), jnp.int32)]
            + [pltpu.VMEM((CH, 128), jnp.float32) for _ in range(R)]
            + [pltpu.VMEM((DIM, CH), jnp.float32) for _ in range(NB)]
            + [pltpu.SemaphoreType.DMA for _ in range(R + NB)]
        ),
    )
    def k(table_hbm, idx_hbm, out_hbm, idx_v, *rest):
        gbufs = rest[:R]
        obufs = rest[R:R + NB]
        gsems = rest[R + NB:2 * R + NB]
        osems = rest[2 * R + NB:2 * R + 2 * NB]
        c = lax.axis_index("c")
        s = lax.axis_index("s")
        wid = s * NC + c
        m0 = wid * NCH
        pltpu.sync_copy(idx_hbm.at[wid], idx_v)
        for b in range(AH):
            pltpu.make_async_copy(
                table_hbm.at[idx_v.at[b]], gbufs[b], gsems[b]).start()

        iota16 = lax.iota(jnp.int32, 16)
        rows_g = [g * 16 + iota16 for g in range(8)]
        perms = [(iota16 + d) & 15 for d in range(16)]

        def transpose_block(gbuf, obuf):
            @plsc.parallel_loop(0, 4, unroll=2)
            def _(fb):
                f0 = fb * 16
                for d in range(16):
                    colv = perms[d] + f0
                    for g in range(8):
                        v = plsc.load_gather(gbuf, [rows_g[g], colv])
                        plsc.store_scatter(obuf, [colv, rows_g[g]], v)

        def consume(j, b, ob):
            m = m0 + j
            t = m // 128
            b0 = (m % 128) * 128
            pltpu.make_async_copy(
                table_hbm.at[idx_v.at[j]], gbufs[b], gsems[b]).wait()

            @pl.when(j + AH < NCH)
            def _():
                pltpu.make_async_copy(
                    table_hbm.at[idx_v.at[j + AH]], gbufs[(b + AH) % R],
                    gsems[(b + AH) % R]).start()

            @pl.when(j >= NB)
            def _():
                mo = m0 + j - NB
                pltpu.make_async_copy(
                    obufs[ob],
                    out_hbm.at[mo // 128, :,
                               pl.ds((mo % 128) * 128, CH)],
                    osems[ob]).wait()
            transpose_block(gbufs[b], obufs[ob])
            pltpu.make_async_copy(
                obufs[ob], out_hbm.at[t, :, pl.ds(b0, CH)], osems[ob]).start()

        nq = NCH // R

        def body2(q, _):
            j0 = q * R
            for b in range(R):
                consume(j0 + b, b, b % NB)
            return 0

        lax.fori_loop(0, nq, body2, 0)
        for ob in range(NB):
            mo = m0 + NCH - NB + ob
            pltpu.make_async_copy(
                obufs[ob % NB],
                out_hbm.at[mo // 128, :, pl.ds((mo % 128) * 128, CH)],
                osems[ob % NB]).wait()

    return k(table_pad, idx)


def kernel(x, table):
    table_pad = jnp.pad(table, ((0, 0), (0, 128 - DIM)))
    idx = x.T.reshape(NW, NCH, CH).astype(jnp.int32)
    out3 = _sc_gather(table_pad, idx)
    return jnp.transpose(out3, (2, 0, 1))
